# baseline (speedup 1.0000x reference)
import math

import jax
import jax.numpy as jnp
from jax import lax
from jax.experimental import pallas as pl
from jax.experimental.pallas import tpu as pltpu
from jax.experimental.pallas import tpu_sc as plsc

_T = 10.0
_NUM = 10
_ALPHA = 0.05
_N = 2048
_F = 384
_K = 64
_BLK = 512
_KPAD = 16
_SCALE = math.sqrt(_T / _N)


def _topk_body(hf_ref, hft_ref, m1_ref, psi_ref,
               vals_ref, idx_ref, g12_ref, rs_scr, t_scr):
    i = pl.program_id(0)
    ng = pl.num_programs(0)
    hfb = hf_ref[...]
    hft = hft_ref[...]
    rss = jnp.sum(hfb * hfb, axis=1, keepdims=True)
    rnb = 1.0 / jnp.maximum(jnp.sqrt(rss), 1e-12)
    hfnb = hfb * rnb
    css = jnp.sum(hft * hft, axis=0, keepdims=True)
    rnc = 1.0 / jnp.maximum(jnp.sqrt(css), 1e-12)
    hfnt = hft * rnc
    s = jnp.dot(hfnb, hfnt, preferred_element_type=jnp.float32)
    col = lax.broadcasted_iota(jnp.int32, (_BLK, _N), 1)
    rowg = lax.broadcasted_iota(jnp.int32, (_BLK, _N), 0) + i * _BLK
    s = jnp.maximum(s, 0.0)
    s = jnp.where(col == rowg, 0.0, s)
    sorig = s
    colk = 0x7FF - col
    vlist = []
    ilist = []
    for _ in range(_NUM):
        m = jnp.max(s, axis=1, keepdims=True)
        amk = jnp.max(jnp.where(s == m, colk, -1), axis=1, keepdims=True)
        vlist.append(m)
        ilist.append(0x7FF - amk)
        s = jnp.where(colk == amk, -1.0, s)
    selval = jnp.where(s == -1.0, sorig, 0.0)
    dn0 = (((0,), (0,)), ((), ()))
    tcol = lax.dot_general(selval, jnp.ones((_BLK, 1), jnp.float32), dn0,
                           preferred_element_type=jnp.float32)
    selfcol = lax.broadcasted_iota(jnp.int32, (_BLK, 1), 0) + i * _BLK
    vlist.append(jnp.zeros((_BLK, _KPAD - _NUM), jnp.float32))
    ilist.extend([selfcol] * (_KPAD - _NUM))
    vals16 = jnp.concatenate(vlist, axis=1)
    vals_ref[...] = vals16
    idx_ref[...] = jnp.concatenate(ilist, axis=1)
    rs_scr[pl.ds(i * _BLK, _BLK), :] = jnp.sum(vals16, axis=1, keepdims=True)

    @pl.when(i == 0)
    def _():
        t_scr[...] = tcol

    @pl.when(i > 0)
    def _():
        t_scr[...] = t_scr[...] + tcol

    @pl.when(i == ng - 1)
    def _():
        d = 0.5 * (rs_scr[...] + t_scr[...])
        dv = 1.0 / jnp.sqrt(d)
        a1 = _SCALE * m1_ref[...] * dv
        a2 = _SCALE * dv - a1
        psi = psi_ref[...]
        g12_ref[...] = jnp.concatenate([psi * a1, psi * a2], axis=1)


def _topk(hf, m1, psi):
    hft = hf.T
    grid = _N // _BLK
    return pl.pallas_call(
        _topk_body,
        grid=(grid,),
        in_specs=[
            pl.BlockSpec((_BLK, _F), lambda i: (i, 0)),
            pl.BlockSpec((_F, _N), lambda i: (0, 0)),
            pl.BlockSpec((_N, 1), lambda i: (0, 0)),
            pl.BlockSpec((_N, _K), lambda i: (0, 0)),
        ],
        out_specs=[
            pl.BlockSpec((_BLK, _KPAD), lambda i: (i, 0)),
            pl.BlockSpec((_BLK, _KPAD), lambda i: (i, 0)),
            pl.BlockSpec((_N, 2 * _K), lambda i: (0, 0)),
        ],
        out_shape=[
            jax.ShapeDtypeStruct((_N, _KPAD), jnp.float32),
            jax.ShapeDtypeStruct((_N, _KPAD), jnp.int32),
            jax.ShapeDtypeStruct((_N, 2 * _K), jnp.float32),
        ],
        scratch_shapes=[
            pltpu.VMEM((_N, 1), jnp.float32),
            pltpu.VMEM((_N, 1), jnp.float32),
        ],
        interpret=False,
    )(hf, hft, m1, psi)


def _sc_body(vals_hbm, idxf_hbm, g12_hbm, z1_hbm, z2_hbm,
             valsb, idxcb, gbuf, z1b, z2b, sem):
    cid = lax.axis_index("c")
    sid = lax.axis_index("s")
    wid = cid * 16 + sid
    base = wid * 64

    pltpu.sync_copy(vals_hbm.at[pl.ds(base, 64)], valsb)
    pltpu.sync_copy(idxf_hbm.at[wid], idxcb)

    copies = [
        pltpu.async_copy(g12_hbm.at[idxcb.at[c]],
                         gbuf.at[pl.ds(c * 128, 128)], sem)
        for c in range(5)
    ]
    for c in copies:
        c.wait()

    def s4(r, carry):
        v16 = valsb[r]
        acc = [jnp.zeros((16,), jnp.float32) for _ in range(8)]
        for j in range(_NUM):
            w = v16[j]
            e = r * _NUM + j
            for jj in range(8):
                acc[jj] = acc[jj] + w * gbuf[e, pl.ds(jj * 16, 16)]
        for jj in range(4):
            z1b[r, pl.ds(jj * 16, 16)] = acc[jj]
            z2b[r, pl.ds(jj * 16, 16)] = acc[jj + 4]
        return carry

    lax.fori_loop(0, 64, s4, 0)
    pltpu.sync_copy(z1b, z1_hbm.at[pl.ds(base, 64)])
    pltpu.sync_copy(z2b, z2_hbm.at[pl.ds(base, 64)])


def _sparse_sc(vals, idxf, g12):
    mesh = plsc.VectorSubcoreMesh(core_axis_name="c", subcore_axis_name="s")
    f32 = jnp.float32
    kern = pl.kernel(
        _sc_body,
        out_type=[
            jax.ShapeDtypeStruct((_N, _K), f32),
            jax.ShapeDtypeStruct((_N, _K), f32),
        ],
        mesh=mesh,
        scratch_types=[
            pltpu.VMEM((64, 16), f32),
            pltpu.VMEM((5, 128), jnp.int32),
            pltpu.VMEM((640, 128), f32),
            pltpu.VMEM((64, 64), f32),
            pltpu.VMEM((64, 64), f32),
            pltpu.SemaphoreType.DMA,
        ],
        interpret=False,
    )
    return kern(vals, idxf, g12)


def _finish_body(g12_ref, z1_ref, z2_ref, out_ref):
    g12 = g12_ref[...]
    g1 = g12[:, :_K]
    g2 = g12[:, _K:]
    z1 = z1_ref[...]
    z2 = z2_ref[...]
    dn = (((0,), (0,)), ((), ()))
    u = lax.dot_general(g1, z2, dn, preferred_element_type=jnp.float32,
                        precision=lax.Precision.HIGHEST)
    v = lax.dot_general(g2, z1, dn, preferred_element_type=jnp.float32,
                        precision=lax.Precision.HIGHEST)
    r = 0.5 * (u + v.T)
    ii = lax.broadcasted_iota(jnp.int32, (_K, _K), 0)
    jj = lax.broadcasted_iota(jnp.int32, (_K, _K), 1)
    diag = jnp.where(ii == jj, r, 0.0)
    tr = jnp.sum(diag)
    total = jnp.sum(r * r)
    dd = jnp.sum(diag * diag)
    loss = -tr / float(_K)
    reg = (total - dd) / float(_K) / 2.0 * _ALPHA
    row = lax.broadcasted_iota(jnp.int32, (8, 128), 0)
    colo = lax.broadcasted_iota(jnp.int32, (8, 128), 1)
    out = jnp.where((row == 0) & (colo == 0), loss,
                    jnp.where((row == 0) & (colo == 1), reg, 0.0))
    out_ref[...] = out


def _finish(g12, z1, z2):
    return pl.pallas_call(
        _finish_body,
        out_shape=jax.ShapeDtypeStruct((8, 128), jnp.float32),
        interpret=False,
    )(g12, z1, z2)


def kernel(lowlevel_feature, midlevel_feature, highlevel_feature, Psi, im):
    hf = highlevel_feature.reshape(-1, highlevel_feature.shape[-1])
    perm = jax.random.permutation(jax.random.key(1), _N)
    m1 = jnp.zeros((_N,), jnp.float32).at[perm[: _N // 2]].set(1.0)
    psi = Psi.reshape(-1, _K)

    vals, idx, g12 = _topk(hf, m1.reshape(_N, 1), psi)

    idxf = idx[:, :_NUM].reshape(32, 5, 128)
    z1, z2 = _sparse_sc(vals, idxf, g12)
    out = _finish(g12, z1, z2)
    return out[0, :2]


# speedup vs baseline: 1.0345x; 1.0345x over previous
import math

import jax
import jax.numpy as jnp
from jax import lax
from jax.experimental import pallas as pl
from jax.experimental.pallas import tpu as pltpu
from jax.experimental.pallas import tpu_sc as plsc

_T = 10.0
_NUM = 10
_ALPHA = 0.05
_N = 2048
_F = 384
_K = 64
_BLK = 512
_KPAD = 16
_SCALE = math.sqrt(_T / _N)


def _topk_body(hf_ref, hfall_ref, m1_ref, psi_ref,
               vals_ref, idx_ref, g12_ref, rs_scr, t_scr):
    i = pl.program_id(0)
    ng = pl.num_programs(0)
    hfb = hf_ref[...]
    hfall = hfall_ref[...]
    rss = jnp.sum(hfb * hfb, axis=1, keepdims=True)
    rnb = 1.0 / jnp.maximum(jnp.sqrt(rss), 1e-12)
    hfnb = hfb * rnb
    css = jnp.sum(hfall * hfall, axis=1, keepdims=True)
    rnc = 1.0 / jnp.maximum(jnp.sqrt(css), 1e-12)
    hfnall = hfall * rnc
    s = lax.dot_general(hfnb, hfnall, (((1,), (1,)), ((), ())),
                        preferred_element_type=jnp.float32)
    col = lax.broadcasted_iota(jnp.int32, (_BLK, _N), 1)
    rowg = lax.broadcasted_iota(jnp.int32, (_BLK, _N), 0) + i * _BLK
    s = jnp.maximum(s, 0.0)
    s = jnp.where(col == rowg, 0.0, s)
    sorig = s
    colk = 0x7FF - col
    vlist = []
    ilist = []
    for _ in range(_NUM):
        m = jnp.max(s, axis=1, keepdims=True)
        amk = jnp.max(jnp.where(s == m, colk, -1), axis=1, keepdims=True)
        vlist.append(m)
        ilist.append(0x7FF - amk)
        s = jnp.where(colk == amk, -1.0, s)
    selval = jnp.where(s == -1.0, sorig, 0.0)
    dn0 = (((0,), (0,)), ((), ()))
    tcol = lax.dot_general(selval, jnp.ones((_BLK, 1), jnp.float32), dn0,
                           preferred_element_type=jnp.float32)
    selfcol = lax.broadcasted_iota(jnp.int32, (_BLK, 1), 0) + i * _BLK
    vlist.append(jnp.zeros((_BLK, _KPAD - _NUM), jnp.float32))
    ilist.extend([selfcol] * (_KPAD - _NUM))
    vals16 = jnp.concatenate(vlist, axis=1)
    vals_ref[...] = vals16
    idx_ref[...] = jnp.concatenate(ilist, axis=1)
    rs_scr[pl.ds(i * _BLK, _BLK), :] = jnp.sum(vals16, axis=1, keepdims=True)

    @pl.when(i == 0)
    def _():
        t_scr[...] = tcol

    @pl.when(i > 0)
    def _():
        t_scr[...] = t_scr[...] + tcol

    @pl.when(i == ng - 1)
    def _():
        d = 0.5 * (rs_scr[...] + t_scr[...])
        dv = 1.0 / jnp.sqrt(d)
        a1 = _SCALE * m1_ref[...] * dv
        a2 = _SCALE * dv - a1
        psi = psi_ref[...]
        g12_ref[...] = jnp.concatenate([psi * a1, psi * a2], axis=1)


def _topk(hf, m1, psi):
    grid = _N // _BLK
    return pl.pallas_call(
        _topk_body,
        grid=(grid,),
        in_specs=[
            pl.BlockSpec((_BLK, _F), lambda i: (i, 0)),
            pl.BlockSpec((_N, _F), lambda i: (0, 0)),
            pl.BlockSpec((_N, 1), lambda i: (0, 0)),
            pl.BlockSpec((_N, _K), lambda i: (0, 0)),
        ],
        out_specs=[
            pl.BlockSpec((_BLK, _KPAD), lambda i: (i, 0)),
            pl.BlockSpec((_BLK, _KPAD), lambda i: (i, 0)),
            pl.BlockSpec((_N, 2 * _K), lambda i: (0, 0)),
        ],
        out_shape=[
            jax.ShapeDtypeStruct((_N, _KPAD), jnp.float32),
            jax.ShapeDtypeStruct((_N, _KPAD), jnp.int32),
            jax.ShapeDtypeStruct((_N, 2 * _K), jnp.float32),
        ],
        scratch_shapes=[
            pltpu.VMEM((_N, 1), jnp.float32),
            pltpu.VMEM((_N, 1), jnp.float32),
        ],
        interpret=False,
    )(hf, hf, m1, psi)


def _sc_body(vals_hbm, idxf_hbm, g12_hbm, z1_hbm, z2_hbm,
             valsb, idxcb, gbuf, z1b, z2b, *sems):
    cid = lax.axis_index("c")
    sid = lax.axis_index("s")
    wid = cid * 16 + sid
    base = wid * 64

    pltpu.sync_copy(vals_hbm.at[pl.ds(base, 64)], valsb)
    pltpu.sync_copy(idxf_hbm.at[wid], idxcb)

    copies = [
        pltpu.async_copy(g12_hbm.at[idxcb.at[c]],
                         gbuf.at[pl.ds(c * 128, 128)], sems[c])
        for c in range(5)
    ]

    def s4(r, carry):
        v16 = valsb[r]
        acc = [jnp.zeros((16,), jnp.float32) for _ in range(8)]
        for j in range(_NUM):
            w = v16[j]
            e = r * _NUM + j
            for jj in range(8):
                acc[jj] = acc[jj] + w * gbuf[e, pl.ds(jj * 16, 16)]
        for jj in range(4):
            z1b[r, pl.ds(jj * 16, 16)] = acc[jj]
            z2b[r, pl.ds(jj * 16, 16)] = acc[jj + 4]
        return carry

    bounds = [0, 12, 25, 38, 51, 64]
    for c in range(5):
        copies[c].wait()
        lax.fori_loop(bounds[c], bounds[c + 1], s4, 0)
    pltpu.sync_copy(z1b, z1_hbm.at[pl.ds(base, 64)])
    pltpu.sync_copy(z2b, z2_hbm.at[pl.ds(base, 64)])


def _sparse_sc(vals, idxf, g12):
    mesh = plsc.VectorSubcoreMesh(core_axis_name="c", subcore_axis_name="s")
    f32 = jnp.float32
    kern = pl.kernel(
        _sc_body,
        out_type=[
            jax.ShapeDtypeStruct((_N, _K), f32),
            jax.ShapeDtypeStruct((_N, _K), f32),
        ],
        mesh=mesh,
        scratch_types=[
            pltpu.VMEM((64, 16), f32),
            pltpu.VMEM((5, 128), jnp.int32),
            pltpu.VMEM((640, 128), f32),
            pltpu.VMEM((64, 64), f32),
            pltpu.VMEM((64, 64), f32),
            pltpu.SemaphoreType.DMA,
            pltpu.SemaphoreType.DMA,
            pltpu.SemaphoreType.DMA,
            pltpu.SemaphoreType.DMA,
            pltpu.SemaphoreType.DMA,
        ],
        interpret=False,
    )
    return kern(vals, idxf, g12)


def _finish_body(g12_ref, z1_ref, z2_ref, out_ref):
    g12 = g12_ref[...]
    g1 = g12[:, :_K]
    g2 = g12[:, _K:]
    z1 = z1_ref[...]
    z2 = z2_ref[...]
    dn = (((0,), (0,)), ((), ()))
    u = lax.dot_general(g1, z2, dn, preferred_element_type=jnp.float32,
                        precision=lax.Precision.HIGHEST)
    v = lax.dot_general(g2, z1, dn, preferred_element_type=jnp.float32,
                        precision=lax.Precision.HIGHEST)
    r = 0.5 * (u + v.T)
    ii = lax.broadcasted_iota(jnp.int32, (_K, _K), 0)
    jj = lax.broadcasted_iota(jnp.int32, (_K, _K), 1)
    diag = jnp.where(ii == jj, r, 0.0)
    tr = jnp.sum(diag)
    total = jnp.sum(r * r)
    dd = jnp.sum(diag * diag)
    loss = -tr / float(_K)
    reg = (total - dd) / float(_K) / 2.0 * _ALPHA
    row = lax.broadcasted_iota(jnp.int32, (8, 128), 0)
    colo = lax.broadcasted_iota(jnp.int32, (8, 128), 1)
    out = jnp.where((row == 0) & (colo == 0), loss,
                    jnp.where((row == 0) & (colo == 1), reg, 0.0))
    out_ref[...] = out


def _finish(g12, z1, z2):
    return pl.pallas_call(
        _finish_body,
        out_shape=jax.ShapeDtypeStruct((8, 128), jnp.float32),
        interpret=False,
    )(g12, z1, z2)


def kernel(lowlevel_feature, midlevel_feature, highlevel_feature, Psi, im):
    hf = highlevel_feature.reshape(-1, highlevel_feature.shape[-1])
    perm = jax.random.permutation(jax.random.key(1), _N)
    m1 = jnp.zeros((_N,), jnp.float32).at[perm[: _N // 2]].set(1.0)
    psi = Psi.reshape(-1, _K)

    vals, idx, g12 = _topk(hf, m1.reshape(_N, 1), psi)

    idxf = idx[:, :_NUM].reshape(32, 5, 128)
    z1, z2 = _sparse_sc(vals, idxf, g12)
    out = _finish(g12, z1, z2)
    return out[0, :2]


# speedup vs baseline: 1.0441x; 1.0093x over previous
import math

import jax
import jax.numpy as jnp
from jax import lax
from jax.experimental import pallas as pl
from jax.experimental.pallas import tpu as pltpu
from jax.experimental.pallas import tpu_sc as plsc

_T = 10.0
_NUM = 10
_ALPHA = 0.05
_N = 2048
_F = 384
_K = 64
_BLK = 512
_KPAD = 16
_SCALE = math.sqrt(_T / _N)


def _topk_body(hf_ref, hfall_ref, m1_ref, psi_ref,
               vals_ref, idx_ref, g12_ref, rs_scr, t_scr):
    i = pl.program_id(0)
    ng = pl.num_programs(0)
    hfb = hf_ref[...]
    hfall = hfall_ref[...]
    rss = jnp.sum(hfb * hfb, axis=1, keepdims=True)
    rnb = 1.0 / jnp.maximum(jnp.sqrt(rss), 1e-12)
    hfnb = hfb * rnb
    css = jnp.sum(hfall * hfall, axis=1, keepdims=True)
    rnc = 1.0 / jnp.maximum(jnp.sqrt(css), 1e-12)
    hfnall = hfall * rnc
    s = lax.dot_general(hfnb, hfnall, (((1,), (1,)), ((), ())),
                        preferred_element_type=jnp.float32)
    col = lax.broadcasted_iota(jnp.int32, (_BLK, _N), 1)
    rowg = lax.broadcasted_iota(jnp.int32, (_BLK, _N), 0) + i * _BLK
    s = jnp.maximum(s, 0.0)
    s = jnp.where(col == rowg, 0.0, s)
    sorig = s
    colk = 0x7FF - col
    vlist = []
    ilist = []
    for _ in range(_NUM):
        m = jnp.max(s, axis=1, keepdims=True)
        amk = jnp.max(jnp.where(s == m, colk, -1), axis=1, keepdims=True)
        vlist.append(m)
        ilist.append(0x7FF - amk)
        s = jnp.where(colk == amk, -1.0, s)
    selval = jnp.where(s == -1.0, sorig, 0.0)
    dn0 = (((0,), (0,)), ((), ()))
    tcol = lax.dot_general(selval, jnp.ones((_BLK, 1), jnp.float32), dn0,
                           preferred_element_type=jnp.float32)
    selfcol = lax.broadcasted_iota(jnp.int32, (_BLK, 1), 0) + i * _BLK
    vlist.append(jnp.zeros((_BLK, _KPAD - _NUM), jnp.float32))
    ilist.extend([selfcol] * (_KPAD - _NUM))
    vals16 = jnp.concatenate(vlist, axis=1)
    vals_ref[...] = vals16
    idx_ref[...] = jnp.concatenate(ilist, axis=1)
    rs_scr[pl.ds(i * _BLK, _BLK), :] = jnp.sum(vals16, axis=1, keepdims=True)

    @pl.when(i == 0)
    def _():
        t_scr[...] = tcol

    @pl.when(i > 0)
    def _():
        t_scr[...] = t_scr[...] + tcol

    @pl.when(i == ng - 1)
    def _():
        d = 0.5 * (rs_scr[...] + t_scr[...])
        dv = 1.0 / jnp.sqrt(d)
        a1 = _SCALE * m1_ref[...] * dv
        a2 = _SCALE * dv - a1
        psi = psi_ref[...]
        g12_ref[...] = jnp.concatenate([psi * a1, psi * a2], axis=1)


def _topk(hf, m1, psi):
    grid = _N // _BLK
    return pl.pallas_call(
        _topk_body,
        grid=(grid,),
        in_specs=[
            pl.BlockSpec((_BLK, _F), lambda i: (i, 0)),
            pl.BlockSpec((_N, _F), lambda i: (0, 0)),
            pl.BlockSpec((_N, 1), lambda i: (0, 0)),
            pl.BlockSpec((_N, _K), lambda i: (0, 0)),
        ],
        out_specs=[
            pl.BlockSpec((_BLK, _KPAD), lambda i: (i, 0)),
            pl.BlockSpec((_BLK, _KPAD), lambda i: (i, 0)),
            pl.BlockSpec((_N, 2 * _K), lambda i: (0, 0)),
        ],
        out_shape=[
            jax.ShapeDtypeStruct((_N, _KPAD), jnp.float32),
            jax.ShapeDtypeStruct((_N, _KPAD), jnp.int32),
            jax.ShapeDtypeStruct((_N, 2 * _K), jnp.float32),
        ],
        scratch_shapes=[
            pltpu.VMEM((_N, 1), jnp.float32),
            pltpu.VMEM((_N, 1), jnp.float32),
        ],
        interpret=False,
    )(hf, hf, m1, psi)


def _sc_body(vals_hbm, idxf_hbm, g12_hbm, z1_hbm, z2_hbm,
             valsb, idxcb, gbuf, z1b, z2b, *sems):
    cid = lax.axis_index("c")
    sid = lax.axis_index("s")
    wid = cid * 16 + sid
    base = wid * 64

    pltpu.sync_copy(vals_hbm.at[pl.ds(base, 64)], valsb)
    pltpu.sync_copy(idxf_hbm.at[wid], idxcb)

    copies = [
        pltpu.async_copy(g12_hbm.at[idxcb.at[c]],
                         gbuf.at[pl.ds(c * 128, 128)], sems[c])
        for c in range(5)
    ]

    def s4(r, carry):
        v16 = valsb[r]
        acc = [jnp.zeros((16,), jnp.float32) for _ in range(8)]
        for j in range(_NUM):
            w = v16[j]
            e = r * _NUM + j
            for jj in range(8):
                acc[jj] = acc[jj] + w * gbuf[e, pl.ds(jj * 16, 16)]
        for jj in range(4):
            z1b[r, pl.ds(jj * 16, 16)] = acc[jj]
            z2b[r, pl.ds(jj * 16, 16)] = acc[jj + 4]
        return carry

    bounds = [0, 12, 25, 38, 51, 64]
    for c in range(5):
        copies[c].wait()
        lax.fori_loop(bounds[c], bounds[c + 1], s4, 0)
    pltpu.sync_copy(z1b, z1_hbm.at[pl.ds(base, 64)])
    pltpu.sync_copy(z2b, z2_hbm.at[pl.ds(base, 64)])


def _sparse_sc(vals, idxf, g12):
    mesh = plsc.VectorSubcoreMesh(core_axis_name="c", subcore_axis_name="s")
    f32 = jnp.float32
    kern = pl.kernel(
        _sc_body,
        out_type=[
            jax.ShapeDtypeStruct((_N, _K), f32),
            jax.ShapeDtypeStruct((_N, _K), f32),
        ],
        mesh=mesh,
        scratch_types=[
            pltpu.VMEM((64, 16), f32),
            pltpu.VMEM((5, 128), jnp.int32),
            pltpu.VMEM((640, 128), f32),
            pltpu.VMEM((64, 64), f32),
            pltpu.VMEM((64, 64), f32),
            pltpu.SemaphoreType.DMA,
            pltpu.SemaphoreType.DMA,
            pltpu.SemaphoreType.DMA,
            pltpu.SemaphoreType.DMA,
            pltpu.SemaphoreType.DMA,
        ],
        interpret=False,
    )
    return kern(vals, idxf, g12)


def _finish_body(g12_ref, z1_ref, z2_ref, out_ref):
    g12 = g12_ref[...]
    g1 = g12[:, :_K]
    g2 = g12[:, _K:]
    z1 = z1_ref[...]
    z2 = z2_ref[...]
    dn = (((0,), (0,)), ((), ()))
    u = lax.dot_general(g1, z2, dn, preferred_element_type=jnp.float32,
                        precision=lax.Precision.HIGHEST)
    v = lax.dot_general(g2, z1, dn, preferred_element_type=jnp.float32,
                        precision=lax.Precision.HIGHEST)
    r = 0.5 * (u + v.T)
    ii = lax.broadcasted_iota(jnp.int32, (_K, _K), 0)
    jj = lax.broadcasted_iota(jnp.int32, (_K, _K), 1)
    diag = jnp.where(ii == jj, r, 0.0)
    tr = jnp.sum(diag)
    total = jnp.sum(r * r)
    dd = jnp.sum(diag * diag)
    loss = -tr / float(_K)
    reg = (total - dd) / float(_K) / 2.0 * _ALPHA
    colo = lax.broadcasted_iota(jnp.int32, (1, 2), 1)
    out_ref[...] = jnp.where(colo == 0, loss, reg)


def _finish(g12, z1, z2):
    return pl.pallas_call(
        _finish_body,
        out_shape=jax.ShapeDtypeStruct((1, 2), jnp.float32),
        interpret=False,
    )(g12, z1, z2)


def kernel(lowlevel_feature, midlevel_feature, highlevel_feature, Psi, im):
    hf = highlevel_feature.reshape(-1, highlevel_feature.shape[-1])
    perm = jax.random.permutation(jax.random.key(1), _N)
    m1 = jnp.zeros((_N,), jnp.float32).at[perm[: _N // 2]].set(1.0)
    psi = Psi.reshape(-1, _K)

    vals, idx, g12 = _topk(hf, m1.reshape(_N, 1), psi)

    idxf = idx[:, :_NUM].reshape(32, 5, 128)
    z1, z2 = _sparse_sc(vals, idxf, g12)
    out = _finish(g12, z1, z2)
    return out.reshape(2)
